# baseline (speedup 1.0000x reference)
import functools

import jax
import jax.numpy as jnp
from jax import lax
from jax.experimental import pallas as pl
from jax.experimental.pallas import tpu as pltpu
from jax.experimental.pallas import tpu_sc as plsc

_B = 16384
_GMF = 32
_MLPD = 32
_TD = 64


_NC = 2
_NS = 16
_NW = _NC * _NS
_BPW = _B // _NW
_HALF = _BPW // 2
_CH = 128


def _sc_gather_body(users_h, items_h, ug_h, ig_h, um_h, im_h, ut_h, it_h,
                    o_ug, o_ig, o_um, o_im, o_ut, o_it,
                    uidx_v, iidx_v, b_ug, b_ig, b_um, b_im, b_ut, b_it, sem):
    wid = lax.axis_index("s") * _NC + lax.axis_index("c")
    base = wid * _BPW
    pltpu.sync_copy(users_h.at[pl.ds(base, _BPW)], uidx_v)
    pltpu.sync_copy(items_h.at[pl.ds(base, _BPW)], iidx_v)
    for h in range(2):
        off = h * _HALF
        copies = []
        for j in range(_HALF // _CH):
            src = pl.ds(off + j * _CH, _CH)
            dst = pl.ds(j * _CH, _CH)
            copies.append(pltpu.async_copy(
                ug_h.at[uidx_v.at[src]], b_ug.at[dst], sem))
            copies.append(pltpu.async_copy(
                ig_h.at[iidx_v.at[src]], b_ig.at[dst], sem))
            copies.append(pltpu.async_copy(
                um_h.at[uidx_v.at[src]], b_um.at[dst], sem))
            copies.append(pltpu.async_copy(
                im_h.at[iidx_v.at[src]], b_im.at[dst], sem))
            copies.append(pltpu.async_copy(
                ut_h.at[uidx_v.at[src]], b_ut.at[dst], sem))
            copies.append(pltpu.async_copy(
                it_h.at[iidx_v.at[src]], b_it.at[dst], sem))
        for c in copies:
            c.wait()
        dst = pl.ds(base + off, _HALF)
        pltpu.sync_copy(b_ug, o_ug.at[dst])
        pltpu.sync_copy(b_ig, o_ig.at[dst])
        pltpu.sync_copy(b_um, o_um.at[dst])
        pltpu.sync_copy(b_im, o_im.at[dst])
        pltpu.sync_copy(b_ut, o_ut.at[dst])
        pltpu.sync_copy(b_it, o_it.at[dst])


def _sc_gather(users, items, ug, ig, um, im, ut, it):
    mesh = plsc.VectorSubcoreMesh(core_axis_name="c", subcore_axis_name="s")
    f32 = jnp.float32
    out_type = (
        jax.ShapeDtypeStruct((_B, _GMF), f32),
        jax.ShapeDtypeStruct((_B, _GMF), f32),
        jax.ShapeDtypeStruct((_B, _MLPD), f32),
        jax.ShapeDtypeStruct((_B, _MLPD), f32),
        jax.ShapeDtypeStruct((_B, _TD), f32),
        jax.ShapeDtypeStruct((_B, _TD), f32),
    )
    scratch = [
        pltpu.VMEM((_BPW,), jnp.int32),
        pltpu.VMEM((_BPW,), jnp.int32),
        pltpu.VMEM((_HALF, _GMF), f32),
        pltpu.VMEM((_HALF, _GMF), f32),
        pltpu.VMEM((_HALF, _MLPD), f32),
        pltpu.VMEM((_HALF, _MLPD), f32),
        pltpu.VMEM((_HALF, _TD), f32),
        pltpu.VMEM((_HALF, _TD), f32),
        pltpu.SemaphoreType.DMA,
    ]
    fn = functools.partial(
        pl.kernel, mesh=mesh, out_type=out_type, scratch_types=scratch,
        compiler_params=pltpu.CompilerParams(use_tc_tiling_on_sc=False),
    )(_sc_gather_body)
    return fn(users, items, ug, ig, um, im, ut, it)


_BM = 2048


def _tc_body(gu, gi, mu, mi, tu, ti, ulg, ulb, ilg, ilb,
             W1, b1, W2, b2, W3, b3, W4, b4, wo, bo, out):
    def ln(x, g, b):
        m = jnp.mean(x, axis=-1, keepdims=True)
        v = jnp.mean((x - m) * (x - m), axis=-1, keepdims=True)
        return (x - m) * lax.rsqrt(v + 1e-5) * g[None, :] + b[None, :]

    tun = ln(tu[...], ulg[...], ulb[...])
    tin = ln(ti[...], ilg[...], ilb[...])
    h = jnp.concatenate([mu[...], mi[...], tun, tin], axis=-1)
    h = jnp.maximum(jnp.dot(h, W1[...],
                            preferred_element_type=jnp.float32) + b1[...][None, :], 0.0)
    h = jnp.maximum(jnp.dot(h, W2[...],
                            preferred_element_type=jnp.float32) + b2[...][None, :], 0.0)
    h = jnp.maximum(jnp.dot(h, W3[...],
                            preferred_element_type=jnp.float32) + b3[...][None, :], 0.0)
    h = jnp.maximum(jnp.dot(h, W4[...],
                            preferred_element_type=jnp.float32) + b4[...][None, :], 0.0)
    fused = jnp.concatenate([gu[...] * gi[...], h], axis=-1)
    out[...] = (jnp.sum(fused * wo[...][None, :], axis=-1, keepdims=True)
                + bo[...][0])


def _tc_mlp(gu, gi, mu, mi, tu, ti, ulg, ulb, ilg, ilb,
            W1, b1, W2, b2, W3, b3, W4, b4, wo, bo, interpret=False):
    grid = (_B // _BM,)

    def row_spec(d):
        return pl.BlockSpec((_BM, d), lambda i: (i, 0))

    def full_spec(shape):
        nd = len(shape)
        return pl.BlockSpec(shape, lambda i: (0,) * nd)

    in_specs = [
        row_spec(_GMF), row_spec(_GMF), row_spec(_MLPD), row_spec(_MLPD),
        row_spec(_TD), row_spec(_TD),
        full_spec((_TD,)), full_spec((_TD,)), full_spec((_TD,)), full_spec((_TD,)),
        full_spec((192, 512)), full_spec((512,)),
        full_spec((512, 256)), full_spec((256,)),
        full_spec((256, 128)), full_spec((128,)),
        full_spec((128, 64)), full_spec((64,)),
        full_spec((96,)), full_spec((1,)),
    ]
    out = pl.pallas_call(
        _tc_body,
        grid=grid,
        in_specs=in_specs,
        out_specs=pl.BlockSpec((_BM, 1), lambda i: (i, 0)),
        out_shape=jax.ShapeDtypeStruct((_B, 1), jnp.float32),
        interpret=interpret,
    )(gu, gi, mu, mi, tu, ti, ulg, ulb, ilg, ilb,
      W1, b1, W2, b2, W3, b3, W4, b4, wo, bo)
    return out[:, 0]


def kernel(users, items, user_id_gmf, item_id_gmf, user_id_mlp, item_id_mlp,
           user_text, item_text, user_ln_g, user_ln_b, item_ln_g, item_ln_b,
           W1, b1, W2, b2, W3, b3, W4, b4, Wo, bo):
    gu, gi, mu, mi, tu, ti = _sc_gather(
        users, items, user_id_gmf, item_id_gmf, user_id_mlp, item_id_mlp,
        user_text, item_text)
    return _tc_mlp(gu, gi, mu, mi, tu, ti,
                   user_ln_g, user_ln_b, item_ln_g, item_ln_b,
                   W1, b1, W2, b2, W3, b3, W4, b4, Wo[:, 0], bo)
